# baseline (speedup 1.0000x reference)
import functools

import jax
import jax.numpy as jnp
from jax import lax
from jax.experimental import pallas as pl
from jax.experimental.pallas import tpu as pltpu
from jax.experimental.pallas import tpu_sc as plsc

VOCAB = 100000
D = 64
B = 4096
L = 200
N = B * L

NC = 2
NS = 16
NW = NC * NS
BW = B // NW
DB = D // 8
BLK = 4


def _sc_kernel(tok_hbm, table_hbm, pos_hbm, out_hbm,
               idx_v, gbuf, obuf, pos_v, sem_g, sem_s0, sem_s1):
    wid = lax.axis_index("s") * NC + lax.axis_index("c")

    pltpu.sync_copy(pos_hbm, pos_v)
    pltpu.sync_copy(tok_hbm.at[:, pl.ds(wid * BW, BW)], idx_v)

    def fire_block(l0):
        half = (l0 // BLK) % 2
        for k in range(BLK):
            pltpu.async_copy(table_hbm.at[idx_v.at[l0 + k]],
                             gbuf.at[pl.ds((half * BLK + k) * BW, BW)],
                             sem_g)

    def wait_block():
        for _ in range(BLK):
            pltpu.make_async_copy(table_hbm.at[idx_v.at[0]],
                                  gbuf.at[pl.ds(0, BW)], sem_g).wait()

    def wait_scatter(sem):
        pltpu.make_async_copy(obuf.at[pl.ds(0, DB)],
                              out_hbm.at[0, :, wid], sem).wait()

    fire_block(0)

    iota = lax.iota(jnp.int32, 16)

    def body(l, carry):
        @pl.when(l % BLK == 0)
        def _():
            wait_block()

            @pl.when(l + BLK < L)
            def _():
                fire_block(l + BLK)

        par = l % 2

        @pl.when((l >= 2) & (par == 0))
        def _():
            wait_scatter(sem_s0)

        @pl.when((l >= 2) & (par == 1))
        def _():
            wait_scatter(sem_s1)

        row0 = (((l // BLK) % 2) * BLK + (l % BLK)) * BW
        lanes = jnp.full((16,), l, jnp.int32)
        rowsv = [row0 + cb * 16 + iota for cb in range(8)]
        cvecs = [cb * 16 + iota for cb in range(8)]

        @plsc.parallel_loop(0, D // 16, 1, unroll=4)
        def db_body(db):
            for k in range(16):
                dvec = db * 16 + ((iota + k) & 15)
                p = plsc.load_gather(pos_v, [lanes, dvec])
                rv = (dvec >> 3) + par * DB
                ddv = dvec & 7
                vals = [plsc.load_gather(gbuf, [rowsv[cb], dvec])
                        for cb in range(8)]
                for cb in range(8):
                    g = vals[cb]
                    plsc.store_scatter(obuf, [rv, ddv, cvecs[cb]], g + g + p)

        @pl.when(par == 0)
        def _():
            pltpu.async_copy(obuf.at[pl.ds(0, DB)],
                             out_hbm.at[l, :, wid], sem_s0)

        @pl.when(par == 1)
        def _():
            pltpu.async_copy(obuf.at[pl.ds(DB, DB)],
                             out_hbm.at[l, :, wid], sem_s1)

        return carry

    lax.fori_loop(0, L, body, 0, unroll=False)

    wait_scatter(sem_s0)
    wait_scatter(sem_s1)


def kernel(tokens, table, pos_emb):
    tok_t = tokens.T.astype(jnp.int32)
    mesh = plsc.VectorSubcoreMesh(core_axis_name="c", subcore_axis_name="s")
    run = functools.partial(
        pl.kernel,
        mesh=mesh,
        out_type=jax.ShapeDtypeStruct((L, DB, NW, 8, 128), jnp.float32),
        scratch_types=[
            pltpu.VMEM((L, BW), jnp.int32),
            pltpu.VMEM((2 * BLK * BW, D), jnp.float32),
            pltpu.VMEM((2 * DB, 8, 128), jnp.float32),
            pltpu.VMEM((L, D), jnp.float32),
            pltpu.SemaphoreType.DMA,
            pltpu.SemaphoreType.DMA,
            pltpu.SemaphoreType.DMA,
        ],
        compiler_params=pltpu.CompilerParams(use_tc_tiling_on_sc=False,
                                             needs_layout_passes=False,
                                             disable_bounds_checks=True),
    )(_sc_kernel)
    out5 = run(tok_t, table, pos_emb)
    return out5.transpose(2, 4, 0, 1, 3).reshape(B, L, D)


# speedup vs baseline: 2.0005x; 2.0005x over previous
import functools

import jax
import jax.numpy as jnp
from jax import lax
from jax.experimental import pallas as pl
from jax.experimental.pallas import tpu as pltpu
from jax.experimental.pallas import tpu_sc as plsc

VOCAB = 100000
D = 64
B = 4096
L = 200
N = B * L

NC = 2
NS = 16
NW = NC * NS
BW = B // NW
DB = D // 8
BLK = 4


def _sc_kernel(tok_hbm, table_hbm, pos_hbm, out_hbm,
               idx_v, gbuf, obuf, pos_v, sem_g, sem_s):
    wid = lax.axis_index("s") * NC + lax.axis_index("c")

    pltpu.sync_copy(pos_hbm, pos_v)
    pltpu.sync_copy(tok_hbm.at[:, pl.ds(wid * BW, BW)], idx_v)

    def fire_block(l0):
        half = (l0 // BLK) % 2
        for k in range(BLK):
            pltpu.async_copy(table_hbm.at[idx_v.at[l0 + k]],
                             gbuf.at[pl.ds((half * BLK + k) * BW, BW)],
                             sem_g)

    def wait_block():
        for _ in range(BLK):
            pltpu.make_async_copy(table_hbm.at[idx_v.at[0]],
                                  gbuf.at[pl.ds(0, BW)], sem_g).wait()

    def wait_scatter():
        pltpu.make_async_copy(obuf.at[pl.ds(0, DB)],
                              out_hbm.at[0, :, wid], sem_s).wait()

    fire_block(0)

    iota = lax.iota(jnp.int32, 16)

    def body(l, carry):
        @pl.when(l % BLK == 0)
        def _():
            wait_block()

            @pl.when(l + BLK < L)
            def _():
                fire_block(l + BLK)

        @pl.when(l >= 2)
        def _():
            wait_scatter()

        par = l % 2
        row0 = (((l // BLK) % 2) * BLK + (l % BLK)) * BW
        lanes = jnp.full((16,), l, jnp.int32)
        rowsv = [row0 + cb * 16 + iota for cb in range(8)]
        cvecs = [cb * 16 + iota for cb in range(8)]

        @plsc.parallel_loop(0, D // 16, 1, unroll=4)
        def db_body(db):
            for k in range(16):
                dvec = db * 16 + ((iota + k) & 15)
                p = plsc.load_gather(pos_v, [lanes, dvec])
                rv = (dvec >> 3) + par * DB
                ddv = dvec & 7
                vals = [plsc.load_gather(gbuf, [rowsv[cb], dvec])
                        for cb in range(8)]
                for cb in range(8):
                    g = vals[cb]
                    plsc.store_scatter(obuf, [rv, ddv, cvecs[cb]], g + g + p)

        pltpu.async_copy(obuf.at[pl.ds(par * DB, DB)],
                         out_hbm.at[l, :, wid], sem_s)
        return carry

    lax.fori_loop(0, L, body, 0, unroll=False)

    wait_scatter()
    wait_scatter()


def kernel(tokens, table, pos_emb):
    tok_t = tokens.T.astype(jnp.int32)
    mesh = plsc.VectorSubcoreMesh(core_axis_name="c", subcore_axis_name="s")
    run = functools.partial(
        pl.kernel,
        mesh=mesh,
        out_type=jax.ShapeDtypeStruct((L, DB, NW, 8, 128), jnp.float32),
        scratch_types=[
            pltpu.VMEM((L, BW), jnp.int32),
            pltpu.VMEM((2 * BLK * BW, D), jnp.float32),
            pltpu.VMEM((2 * DB, 8, 128), jnp.float32),
            pltpu.VMEM((L, D), jnp.float32),
            pltpu.SemaphoreType.DMA,
            pltpu.SemaphoreType.DMA,
        ],
        compiler_params=pltpu.CompilerParams(use_tc_tiling_on_sc=False,
                                             needs_layout_passes=False,
                                             disable_bounds_checks=True),
    )(_sc_kernel)
    out5 = run(tok_t, table, pos_emb)
    return out5.transpose(2, 4, 0, 1, 3).reshape(B, L, D)
